# Initial kernel scaffold; baseline (speedup 1.0000x reference)
import functools

import jax
import jax.numpy as jnp
from jax import lax
from jax.experimental import pallas as pl
from jax.experimental.pallas import tpu as pltpu
from jax.experimental.pallas import tpu_sc as plsc

_B, _H, _W, _N = 8, 128, 128, 64
_HW = _H * _W
_L = 16
_NVREG = _HW // _L


def _sc_weights_body(prob_hbm, unc_hbm, w_hbm, prob_v, unc_v, w_v):
    c = lax.axis_index("c")
    s = lax.axis_index("s")
    wid = s * 2 + c

    @pl.when(wid < _B)
    def _():
        pltpu.sync_copy(prob_hbm.at[wid], prob_v)
        pltpu.sync_copy(unc_hbm.at[wid], unc_v)

        def cond(carry):
            v, count = carry
            return jnp.logical_and(v < _NVREG, count < _N)

        def body(carry):
            v, count = carry
            p = prob_v[pl.ds(v * _L, _L)]
            m = p > 0.5
            cum = plsc.cumsum(m.astype(jnp.int32))
            rank = count + cum - 1
            ok = jnp.logical_and(m, rank < _N)
            u = unc_v[pl.ds(v * _L, _L)]
            plsc.store_scatter(w_v, [rank], jnp.exp(-u), mask=ok)
            return v + 1, count + jnp.max(cum)

        _, count = lax.while_loop(
            cond, body, (jnp.int32(0), jnp.int32(0)))

        @pl.when(count < _N)
        def _():
            def body2(v, nc):
                p = prob_v[pl.ds(v * _L, _L)]
                m = p <= 0.5
                cum = plsc.cumsum(m.astype(jnp.int32))
                rank = count + nc + cum - 1
                ok = jnp.logical_and(m, rank < _N)
                u = unc_v[pl.ds(v * _L, _L)]
                plsc.store_scatter(w_v, [rank], jnp.exp(-u), mask=ok)
                return nc + jnp.max(cum)

            lax.fori_loop(0, _NVREG, body2, jnp.int32(0))

        pltpu.sync_copy(w_v, w_hbm.at[wid])


@jax.jit
def _sc_weights(prob, unc):
    mesh = plsc.VectorSubcoreMesh(core_axis_name="c", subcore_axis_name="s")
    return pl.kernel(
        _sc_weights_body,
        out_type=jax.ShapeDtypeStruct((_B, _N), jnp.float32),
        mesh=mesh,
        scratch_types=[
            pltpu.VMEM((_HW,), jnp.float32),
            pltpu.VMEM((_HW,), jnp.float32),
            pltpu.VMEM((_N,), jnp.float32),
        ],
    )(prob, unc)


_C = 2048


def _tc_loss_body(x_ref, y_ref, p_ref, tx_ref, ty_ref, w_ref, out_ref):
    b = pl.program_id(0)
    tx = tx_ref[...]
    ty = ty_ref[...]
    inf = jnp.float32(jnp.inf)

    def body(i, acc):
        xc = x_ref[0, pl.ds(i * _C, _C)]
        yc = y_ref[0, pl.ds(i * _C, _C)]
        pc = p_ref[0, pl.ds(i * _C, _C)]
        dx = tx - xc[None, :]
        dy = ty - yc[None, :]
        d2 = dx * dx + dy * dy
        d2 = jnp.where((pc > 0.5)[None, :], d2, inf)
        return jnp.minimum(acc, jnp.min(d2, axis=1, keepdims=True))

    dmin = lax.fori_loop(0, _HW // _C, body, jnp.full((_N, 1), inf))
    loss = jnp.sum(jnp.sqrt(dmin) * w_ref[...]) * (1.0 / (_N * _B))

    @pl.when(b == 0)
    def _():
        out_ref[0, 0] = 0.0

    out_ref[0, 0] += loss


@jax.jit
def _tc_loss(xs, ys, prob, txT, tyT, wT):
    return pl.pallas_call(
        _tc_loss_body,
        grid=(_B,),
        in_specs=[
            pl.BlockSpec((1, _HW), lambda b: (b, 0)),
            pl.BlockSpec((1, _HW), lambda b: (b, 0)),
            pl.BlockSpec((1, _HW), lambda b: (b, 0)),
            pl.BlockSpec((_N, 1), lambda b: (0, b)),
            pl.BlockSpec((_N, 1), lambda b: (0, b)),
            pl.BlockSpec((_N, 1), lambda b: (0, b)),
        ],
        out_specs=pl.BlockSpec((1, 1), lambda b: (0, 0)),
        out_shape=jax.ShapeDtypeStruct((1, 1), jnp.float32),
    )(xs, ys, prob, txT, tyT, wT)


def kernel(loc_pred, uncertainty, true_locations, prob_map):
    loc = loc_pred.reshape(_B, _HW, 2)
    xs = loc[:, :, 0]
    ys = loc[:, :, 1]
    prob = prob_map.reshape(_B, _HW)
    unc = uncertainty.reshape(_B, _HW)
    txT = true_locations[:, :, 0].T
    tyT = true_locations[:, :, 1].T
    w = _sc_weights(prob, unc)
    out = _tc_loss(xs, ys, prob, txT, tyT, w.T)
    return out[0, 0]


# speedup vs baseline: 1.0558x; 1.0558x over previous
import functools

import jax
import jax.numpy as jnp
from jax import lax
from jax.experimental import pallas as pl
from jax.experimental.pallas import tpu as pltpu
from jax.experimental.pallas import tpu_sc as plsc

_B, _H, _W, _N = 8, 128, 128, 64
_HW = _H * _W
_L = 16
_NVREG = _HW // _L


def _sc_weights_body(prob_hbm, unc_hbm, w_hbm, prob_v, unc_v, w_v, cnt_s):
    c = lax.axis_index("c")
    s = lax.axis_index("s")
    wid = s * 2 + c
    active = wid < _B

    @pl.when(active)
    def _():
        pltpu.sync_copy(prob_hbm.at[wid], prob_v)
        pltpu.sync_copy(unc_hbm.at[wid], unc_v)
        cnt_s[0] = 0
        cnt_s[1] = 0

        def body(v, _):
            @pl.when(cnt_s[0] < _N)
            def _():
                count = cnt_s[0]
                p = prob_v[pl.ds(v * _L, _L)]
                m = p > 0.5
                cum = plsc.cumsum(m.astype(jnp.int32))
                rank = count + cum - 1
                ok = jnp.logical_and(m, rank < _N)
                u = unc_v[pl.ds(v * _L, _L)]
                plsc.store_scatter(w_v, [rank], jnp.exp(-u), mask=ok)
                cnt_s[0] = count + jnp.max(cum)

            return 0

        lax.fori_loop(0, _NVREG, body, 0)

        def body2(v, _):
            @pl.when(cnt_s[0] + cnt_s[1] < _N)
            def _():
                base = cnt_s[0] + cnt_s[1]
                p = prob_v[pl.ds(v * _L, _L)]
                m = p <= 0.5
                cum = plsc.cumsum(m.astype(jnp.int32))
                rank = base + cum - 1
                ok = jnp.logical_and(m, rank < _N)
                u = unc_v[pl.ds(v * _L, _L)]
                plsc.store_scatter(w_v, [rank], jnp.exp(-u), mask=ok)
                cnt_s[1] = cnt_s[1] + jnp.max(cum)

            return 0

        lax.fori_loop(0, _NVREG, body2, 0)

        pltpu.sync_copy(w_v, w_hbm.at[wid])


@jax.jit
def _sc_weights(prob, unc):
    mesh = plsc.VectorSubcoreMesh(core_axis_name="c", subcore_axis_name="s")
    return pl.kernel(
        _sc_weights_body,
        out_type=jax.ShapeDtypeStruct((_B, _N), jnp.float32),
        mesh=mesh,
        compiler_params=pltpu.CompilerParams(needs_layout_passes=False),
        scratch_types=[
            pltpu.VMEM((_HW,), jnp.float32),
            pltpu.VMEM((_HW,), jnp.float32),
            pltpu.VMEM((_N,), jnp.float32),
            pltpu.SMEM((2,), jnp.int32),
        ],
    )(prob, unc)


_C = 2048


def _tc_loss_body(x_ref, y_ref, p_ref, tx_ref, ty_ref, w_ref, out_ref):
    b = pl.program_id(0)
    tx = tx_ref[0]
    ty = ty_ref[0]
    inf = jnp.float32(jnp.inf)

    def body(i, acc):
        xc = x_ref[0, 0, pl.ds(i * _C, _C)]
        yc = y_ref[0, 0, pl.ds(i * _C, _C)]
        pc = p_ref[0, 0, pl.ds(i * _C, _C)]
        dx = tx - xc[None, :]
        dy = ty - yc[None, :]
        d2 = dx * dx + dy * dy
        d2 = jnp.where((pc > 0.5)[None, :], d2, inf)
        return jnp.minimum(acc, jnp.min(d2, axis=1, keepdims=True))

    dmin = lax.fori_loop(0, _HW // _C, body, jnp.full((_N, 1), inf))
    loss = jnp.sum(jnp.sqrt(dmin) * w_ref[0]) * (1.0 / (_N * _B))

    @pl.when(b == 0)
    def _():
        out_ref[...] = jnp.zeros((1, 1), jnp.float32)

    out_ref[...] = out_ref[...] + loss


@jax.jit
def _tc_loss(xs, ys, prob, txc, tyc, wc):
    return pl.pallas_call(
        _tc_loss_body,
        grid=(_B,),
        in_specs=[
            pl.BlockSpec((1, 1, _HW), lambda b: (b, 0, 0)),
            pl.BlockSpec((1, 1, _HW), lambda b: (b, 0, 0)),
            pl.BlockSpec((1, 1, _HW), lambda b: (b, 0, 0)),
            pl.BlockSpec((1, _N, 1), lambda b: (b, 0, 0)),
            pl.BlockSpec((1, _N, 1), lambda b: (b, 0, 0)),
            pl.BlockSpec((1, _N, 1), lambda b: (b, 0, 0)),
        ],
        out_specs=pl.BlockSpec((1, 1), lambda b: (0, 0)),
        out_shape=jax.ShapeDtypeStruct((1, 1), jnp.float32),
    )(xs, ys, prob, txc, tyc, wc)


def kernel(loc_pred, uncertainty, true_locations, prob_map):
    loc = loc_pred.reshape(_B, _HW, 2)
    xs = loc[:, :, 0].reshape(_B, 1, _HW)
    ys = loc[:, :, 1].reshape(_B, 1, _HW)
    prob = prob_map.reshape(_B, _HW)
    unc = uncertainty.reshape(_B, _HW)
    txc = true_locations[:, :, 0].reshape(_B, _N, 1)
    tyc = true_locations[:, :, 1].reshape(_B, _N, 1)
    w = _sc_weights(prob, unc)
    out = _tc_loss(xs, ys, prob.reshape(_B, 1, _HW), txc, tyc,
                   w.reshape(_B, _N, 1))
    return out[0, 0]
